# Initial kernel scaffold; baseline (speedup 1.0000x reference)
import jax
import jax.numpy as jnp
from jax.experimental import pallas as pl


def kernel(news_input, table, W, b):
    raise NotImplementedError("write your pallas kernel here")


# speedup vs baseline: 7.0887x; 7.0887x over previous
import functools

import jax
import jax.numpy as jnp
from jax import lax
from jax.experimental import pallas as pl
from jax.experimental.pallas import tpu as pltpu
from jax.experimental.pallas import tpu_sc as plsc

EMB_DIM = 128
BATCH = 4096
SEQ = 50

_NC = 2
_NS = 16
_NW = _NC * _NS

_ROWS_PER_W = BATCH // _NW
_ROWS_PER_CHUNK = 2
_IDX_PER_CHUNK = _ROWS_PER_CHUNK * SEQ
_CHUNKS = _ROWS_PER_W // _ROWS_PER_CHUNK


def _sc_pool_sums(news2, table):
  mesh = plsc.VectorSubcoreMesh(core_axis_name="c", subcore_axis_name="s")

  @functools.partial(
      pl.kernel,
      mesh=mesh,
      out_type=jax.ShapeDtypeStruct((BATCH, EMB_DIM), jnp.float32),
      scratch_types=[
          pltpu.VMEM((_CHUNKS, _IDX_PER_CHUNK), jnp.int32),
          pltpu.VMEM((_IDX_PER_CHUNK, EMB_DIM), jnp.float32),
          pltpu.VMEM((_ROWS_PER_W, EMB_DIM), jnp.float32),
          pltpu.SemaphoreType.DMA,
      ],
  )
  def k(news_hbm, table_hbm, out_hbm, idx_v, g_v, acc_v, sem):
    wid = lax.axis_index("s") * _NC + lax.axis_index("c")
    pltpu.sync_copy(news_hbm.at[pl.ds(wid * _CHUNKS, _CHUNKS)], idx_v)

    def chunk_body(ci, carry):
      pltpu.async_copy(table_hbm.at[idx_v.at[ci]], g_v, sem).wait()
      for r in range(_ROWS_PER_CHUNK):
        def acc_body(l, accs):
          row = r * SEQ + l
          return tuple(
              accs[j] + g_v[row, pl.ds(j * 16, 16)] for j in range(8)
          )
        accs = lax.fori_loop(
            0, SEQ, acc_body,
            tuple(jnp.zeros((16,), jnp.float32) for _ in range(8)))
        out_row = ci * _ROWS_PER_CHUNK + r
        for j in range(8):
          acc_v[out_row, pl.ds(j * 16, 16)] = accs[j]
      return carry

    lax.fori_loop(0, _CHUNKS, chunk_body, 0)
    pltpu.sync_copy(acc_v, out_hbm.at[pl.ds(wid * _ROWS_PER_W, _ROWS_PER_W)])

  return k(news2, table)


def _tc_finish_kernel(news_ref, s_ref, w_ref, b_ref, o_ref):
  cnt = jnp.sum((news_ref[...] != 0).astype(jnp.float32), axis=1,
                keepdims=True)
  vec = s_ref[...] / (cnt + 1e-8)
  out = lax.dot_general(vec, w_ref[...], (((1,), (1,)), ((), ())),
                        preferred_element_type=jnp.float32)
  o_ref[...] = jnp.maximum(out + b_ref[...], 0.0)


def _tc_finish(news_input, sums, W, b):
  blk = 1024
  grid = BATCH // blk
  return pl.pallas_call(
      _tc_finish_kernel,
      out_shape=jax.ShapeDtypeStruct((BATCH, EMB_DIM), jnp.float32),
      grid=(grid,),
      in_specs=[
          pl.BlockSpec((blk, SEQ), lambda i: (i, 0)),
          pl.BlockSpec((blk, EMB_DIM), lambda i: (i, 0)),
          pl.BlockSpec((EMB_DIM, EMB_DIM), lambda i: (0, 0)),
          pl.BlockSpec((1, EMB_DIM), lambda i: (0, 0)),
      ],
      out_specs=pl.BlockSpec((blk, EMB_DIM), lambda i: (i, 0)),
  )(news_input, sums, W, b.reshape(1, EMB_DIM))


def kernel(news_input, table, W, b):
  news2 = news_input.reshape(BATCH // _ROWS_PER_CHUNK, _IDX_PER_CHUNK)
  sums = _sc_pool_sums(news2, table)
  return _tc_finish(news_input, sums, W, b)
